# baseline (speedup 1.0000x reference)
import jax
import jax.numpy as jnp
from jax.experimental import pallas as pl


def _add_body(patch_ref, pos_ref, out_ref):
    out_ref[...] = patch_ref[...] + pos_ref[...]


def kernel(patch, pos_table):
    B, P, D = patch.shape
    CH = 256
    n_chunks = P // CH
    return pl.pallas_call(
        _add_body,
        grid=(n_chunks, B),
        in_specs=[
            pl.BlockSpec((1, CH, D), lambda c, b: (b, c, 0)),
            pl.BlockSpec((CH, D), lambda c, b: (c, 0)),
        ],
        out_specs=pl.BlockSpec((1, CH, D), lambda c, b: (b, c, 0)),
        out_shape=jax.ShapeDtypeStruct(patch.shape, patch.dtype),
    )(patch, pos_table)


# speedup vs baseline: 1.7282x; 1.7282x over previous
import jax
import jax.numpy as jnp
from jax.experimental import pallas as pl


def _add_body(patch_ref, pos_ref, out_ref):
    out_ref[...] = patch_ref[...] + pos_ref[...]


def kernel(patch, pos_table):
    B, P, D = patch.shape
    return pl.pallas_call(
        _add_body,
        grid=(B,),
        in_specs=[
            pl.BlockSpec((1, P, D), lambda b: (b, 0, 0)),
            pl.BlockSpec((P, D), lambda b: (0, 0)),
        ],
        out_specs=pl.BlockSpec((1, P, D), lambda b: (b, 0, 0)),
        out_shape=jax.ShapeDtypeStruct(patch.shape, patch.dtype),
    )(patch, pos_table)
